# baseline (speedup 1.0000x reference)
import functools

import jax
import jax.numpy as jnp
from jax.experimental import pallas as pl
from jax.experimental.pallas import tpu as pltpu

Q = 1024
D = 64
BK = 2048
INT_MAX = jnp.iinfo(jnp.int32).max


def _nn_kernel(qsq_ref, q_ref, k_ref, ksq_ref, out_ref, minval_ref):
    i = pl.program_id(0)

    q = q_ref[...]
    k = k_ref[...]
    m = jax.lax.dot_general(
        q, k, (((1,), (1,)), ((), ())),
        preferred_element_type=jnp.float32)
    d2 = (qsq_ref[...] - 2.0 * m) + ksq_ref[...]

    ids = i * BK + jax.lax.broadcasted_iota(jnp.int32, (1, BK), 1)
    bmin = jnp.min(d2, axis=1, keepdims=True)
    barg = jnp.min(jnp.where(d2 == bmin, ids, INT_MAX),
                   axis=1, keepdims=True)

    @pl.when(i == 0)
    def _init():
        minval_ref[...] = bmin
        out_ref[...] = barg

    @pl.when(i > 0)
    def _merge():
        better = bmin < minval_ref[...]
        minval_ref[...] = jnp.where(better, bmin, minval_ref[...])
        out_ref[...] = jnp.where(better, barg, out_ref[...])


@jax.jit
def kernel(queries, keys):
    n_keys = keys.shape[0]
    n_blocks = pl.cdiv(n_keys, BK)
    k_pad = n_blocks * BK

    qsq = jnp.sum(queries * queries, axis=1, keepdims=True)
    ksq = jnp.sum(keys * keys, axis=1)
    ksq = jnp.pad(ksq, (0, k_pad - n_keys),
                  constant_values=jnp.inf)[None, :]
    keys_p = jnp.pad(keys, ((0, k_pad - n_keys), (0, 0)))

    out = pl.pallas_call(
        _nn_kernel,
        grid=(n_blocks,),
        in_specs=[
            pl.BlockSpec((Q, 1), lambda i: (0, 0)),
            pl.BlockSpec((Q, D), lambda i: (0, 0)),
            pl.BlockSpec((BK, D), lambda i: (i, 0)),
            pl.BlockSpec((1, BK), lambda i: (0, i)),
        ],
        out_specs=pl.BlockSpec((Q, 1), lambda i: (0, 0)),
        out_shape=jax.ShapeDtypeStruct((Q, 1), jnp.int32),
        scratch_shapes=[pltpu.VMEM((Q, 1), jnp.float32)],
    )(qsq, queries, keys_p, ksq)
    return out[:, 0]


# speedup vs baseline: 1.3040x; 1.3040x over previous
import jax
import jax.numpy as jnp
from jax.experimental import pallas as pl
from jax.experimental.pallas import tpu as pltpu

Q = 1024
D = 64
BK = 2048
T = BK // 128
INT_MAX = jnp.iinfo(jnp.int32).max


def _nn_kernel(qsq_ref, qm2_ref, k_ref, ksq_ref, out_ref, val_ref, idx_ref):
    i = pl.program_id(0)
    nb = pl.num_programs(0)

    @pl.when(i == 0)
    def _init():
        val_ref[...] = jnp.full((Q, 128), jnp.inf, dtype=jnp.float32)
        idx_ref[...] = jnp.zeros((Q, 128), dtype=jnp.int32)

    m2 = jax.lax.dot_general(
        qm2_ref[...], k_ref[...], (((1,), (1,)), ((), ())),
        preferred_element_type=jnp.float32)
    d2 = (qsq_ref[...] + m2) + ksq_ref[...]

    val = val_ref[...]
    idx = idx_ref[...]
    for t in range(T):
        v = d2[:, t * 128:(t + 1) * 128]
        tg = i * T + t
        mask = v < val
        val = jnp.minimum(val, v)
        idx = jnp.where(mask, tg, idx)
    val_ref[...] = val
    idx_ref[...] = idx

    @pl.when(i == nb - 1)
    def _finish():
        gmin = jnp.min(val, axis=1, keepdims=True)
        lane = jax.lax.broadcasted_iota(jnp.int32, (Q, 128), 1)
        full_idx = idx * 128 + lane
        out_ref[...] = jnp.min(
            jnp.where(val == gmin, full_idx, INT_MAX),
            axis=1, keepdims=True)


@jax.jit
def kernel(queries, keys):
    n_keys = keys.shape[0]
    n_blocks = pl.cdiv(n_keys, BK)
    k_pad = n_blocks * BK

    qsq = jnp.sum(queries * queries, axis=1, keepdims=True)
    qm2 = -2.0 * queries
    ksq = jnp.sum(keys * keys, axis=1)
    ksq = jnp.pad(ksq, (0, k_pad - n_keys),
                  constant_values=jnp.inf)[None, :]
    keys_p = jnp.pad(keys, ((0, k_pad - n_keys), (0, 0)))

    out = pl.pallas_call(
        _nn_kernel,
        grid=(n_blocks,),
        in_specs=[
            pl.BlockSpec((Q, 1), lambda i: (0, 0)),
            pl.BlockSpec((Q, D), lambda i: (0, 0)),
            pl.BlockSpec((BK, D), lambda i: (i, 0)),
            pl.BlockSpec((1, BK), lambda i: (0, i)),
        ],
        out_specs=pl.BlockSpec((Q, 1), lambda i: (0, 0)),
        out_shape=jax.ShapeDtypeStruct((Q, 1), jnp.int32),
        scratch_shapes=[pltpu.VMEM((Q, 128), jnp.float32),
                        pltpu.VMEM((Q, 128), jnp.int32)],
    )(qsq, qm2, keys_p, ksq)
    return out[:, 0]


# speedup vs baseline: 1.5171x; 1.1634x over previous
import jax
import jax.numpy as jnp
from jax.experimental import pallas as pl
from jax.experimental.pallas import tpu as pltpu

Q = 1024
D = 64
BK = 2048
T = BK // 128
INT_MAX = jnp.iinfo(jnp.int32).max


def _nn_kernel(qsq_ref, qm2_ref, k_ref, ksq_ref, out_ref, val_ref, idx_ref):
    i = pl.program_id(0)
    nb = pl.num_programs(0)

    @pl.when(i == 0)
    def _init():
        val_ref[...] = jnp.full((Q, 128), jnp.inf, dtype=jnp.float32)
        idx_ref[...] = jnp.zeros((Q, 128), dtype=jnp.int32)

    m2 = jax.lax.dot_general(
        qm2_ref[...], k_ref[...], (((1,), (1,)), ((), ())),
        preferred_element_type=jnp.float32)
    d2 = (qsq_ref[...] + m2) + ksq_ref[...]

    val = val_ref[...]
    idx = idx_ref[...]
    for t in range(T):
        v = d2[:, t * 128:(t + 1) * 128]
        tg = i * T + t
        mask = v < val
        val = jnp.where(mask, v, val)
        idx = jnp.where(mask, tg, idx)
    val_ref[...] = val
    idx_ref[...] = idx

    @pl.when(i == nb - 1)
    def _finish():
        gmin = jnp.min(val, axis=1, keepdims=True)
        lane = jax.lax.broadcasted_iota(jnp.int32, (Q, 128), 1)
        full_idx = idx * 128 + lane
        out_ref[...] = jnp.min(
            jnp.where(val == gmin, full_idx, INT_MAX),
            axis=1, keepdims=True)


@jax.jit
def kernel(queries, keys):
    n_keys = keys.shape[0]
    n_blocks = pl.cdiv(n_keys, BK)
    k_pad = n_blocks * BK

    qsq = jnp.sum(queries * queries, axis=1, keepdims=True)
    qm2 = -2.0 * queries
    ksq = jnp.sum(keys * keys, axis=1)
    ksq = jnp.pad(ksq, (0, k_pad - n_keys),
                  constant_values=jnp.inf)[None, :]

    out = pl.pallas_call(
        _nn_kernel,
        grid=(n_blocks,),
        in_specs=[
            pl.BlockSpec((Q, 1), lambda i: (0, 0)),
            pl.BlockSpec((Q, D), lambda i: (0, 0)),
            pl.BlockSpec((BK, D), lambda i: (i, 0)),
            pl.BlockSpec((1, BK), lambda i: (0, i)),
        ],
        out_specs=pl.BlockSpec((Q, 1), lambda i: (0, 0)),
        out_shape=jax.ShapeDtypeStruct((Q, 1), jnp.int32),
        scratch_shapes=[pltpu.VMEM((Q, 128), jnp.float32),
                        pltpu.VMEM((Q, 128), jnp.int32)],
    )(qsq, qm2, keys, ksq)
    return out[:, 0]


# speedup vs baseline: 2.0580x; 1.3565x over previous
import jax
import jax.numpy as jnp
from jax.experimental import pallas as pl
from jax.experimental.pallas import tpu as pltpu

Q = 1024
D = 64
BK = 2048
T = BK // 128
INT_MAX = jnp.iinfo(jnp.int32).max


def _nn_kernel(qsq_ref, qm2_ref, k_ref, ksq_ref, out_ref, val_ref, idx_ref):
    i = pl.program_id(0)
    nb = pl.num_programs(0)

    @pl.when(i == 0)
    def _init():
        val_ref[...] = jnp.full((Q, 128), jnp.inf, dtype=jnp.float32)
        idx_ref[...] = jnp.zeros((Q, 128), dtype=jnp.int32)

    m2 = jax.lax.dot_general(
        qm2_ref[...], k_ref[...], (((1,), (0,)), ((), ())),
        preferred_element_type=jnp.float32)
    d2 = (qsq_ref[...] + m2) + ksq_ref[...]

    val = val_ref[...]
    idx = idx_ref[...]
    for t in range(T):
        v = d2[:, t * 128:(t + 1) * 128]
        tg = i * T + t
        mask = v < val
        val = jnp.where(mask, v, val)
        idx = jnp.where(mask, tg, idx)
    val_ref[...] = val
    idx_ref[...] = idx

    @pl.when(i == nb - 1)
    def _finish():
        gmin = jnp.min(val, axis=1, keepdims=True)
        lane = jax.lax.broadcasted_iota(jnp.int32, (Q, 128), 1)
        full_idx = idx * 128 + lane
        out_ref[...] = jnp.min(
            jnp.where(val == gmin, full_idx, INT_MAX),
            axis=1, keepdims=True)


@jax.jit
def kernel(queries, keys):
    n_keys = keys.shape[0]
    n_blocks = pl.cdiv(n_keys, BK)
    k_pad = n_blocks * BK

    qsq = jnp.sum(queries * queries, axis=1, keepdims=True)
    qm2 = -2.0 * queries
    ksq = jnp.sum(keys * keys, axis=1)
    ksq = jnp.pad(ksq, (0, k_pad - n_keys),
                  constant_values=jnp.inf)[None, :]
    keys_t = keys.T

    out = pl.pallas_call(
        _nn_kernel,
        grid=(n_blocks,),
        in_specs=[
            pl.BlockSpec((Q, 1), lambda i: (0, 0)),
            pl.BlockSpec((Q, D), lambda i: (0, 0)),
            pl.BlockSpec((D, BK), lambda i: (0, i)),
            pl.BlockSpec((1, BK), lambda i: (0, i)),
        ],
        out_specs=pl.BlockSpec((Q, 1), lambda i: (0, 0)),
        out_shape=jax.ShapeDtypeStruct((Q, 1), jnp.int32),
        scratch_shapes=[pltpu.VMEM((Q, 128), jnp.float32),
                        pltpu.VMEM((Q, 128), jnp.int32)],
    )(qsq, qm2, keys_t, ksq)
    return out[:, 0]
